# baseline (speedup 1.0000x reference)
import functools

import jax
import jax.numpy as jnp
from jax.experimental import pallas as pl
from jax.experimental.pallas import tpu as pltpu


def _s1_body(x_ref, w1_ref, s1_ref):
    s1_ref[...] = jnp.dot(x_ref[...], w1_ref[...],
                          preferred_element_type=jnp.float32)


def _pass_b_body(adj_ref, s1_ref, b1_ref, w2_ref, t_ref):
    acc = jnp.dot(adj_ref[...], s1_ref[...],
                  preferred_element_type=jnp.float32)
    h = jnp.maximum(acc + b1_ref[...], 0.0)
    t_ref[...] = jnp.dot(h, w2_ref[...], preferred_element_type=jnp.float32)


def _pass_c_body(adj_ref, t_ref, b2_ref, out_ref):
    o = jnp.dot(adj_ref[...], t_ref[...],
                preferred_element_type=jnp.float32)
    o = o + b2_ref[...]
    m = jnp.max(o, axis=1, keepdims=True)
    u = o - m
    lse = jnp.log(jnp.sum(jnp.exp(u), axis=1, keepdims=True))
    out_ref[...] = u - lse


@functools.partial(jax.jit, static_argnames=())
def kernel(x, adj, W1, b1, W2, b2):
    N, F = x.shape
    H = W1.shape[1]
    C = W2.shape[1]
    BM = 256
    G = pl.cdiv(N, BM)

    b1_2d = b1.reshape(1, H)
    b2_2d = b2.reshape(1, C)

    s1 = pl.pallas_call(
        _s1_body,
        out_shape=jax.ShapeDtypeStruct((N, H), jnp.float32),
    )(x, W1)

    t = pl.pallas_call(
        _pass_b_body,
        grid=(G,),
        in_specs=[
            pl.BlockSpec((BM, N), lambda i: (i, 0)),
            pl.BlockSpec((N, H), lambda i: (0, 0)),
            pl.BlockSpec((1, H), lambda i: (0, 0)),
            pl.BlockSpec((H, C), lambda i: (0, 0)),
        ],
        out_specs=pl.BlockSpec((BM, C), lambda i: (i, 0)),
        out_shape=jax.ShapeDtypeStruct((N, C), jnp.float32),
        compiler_params=pltpu.CompilerParams(
            dimension_semantics=("parallel",)),
    )(adj, s1, b1_2d, W2)

    out = pl.pallas_call(
        _pass_c_body,
        grid=(G,),
        in_specs=[
            pl.BlockSpec((BM, N), lambda i: (i, 0)),
            pl.BlockSpec((N, C), lambda i: (0, 0)),
            pl.BlockSpec((1, C), lambda i: (0, 0)),
        ],
        out_specs=pl.BlockSpec((BM, C), lambda i: (i, 0)),
        out_shape=jax.ShapeDtypeStruct((N, C), jnp.float32),
        compiler_params=pltpu.CompilerParams(
            dimension_semantics=("parallel",)),
    )(adj, t, b2_2d)

    return out
